# baseline (speedup 1.0000x reference)
import jax
import jax.numpy as jnp
from jax.experimental import pallas as pl
from jax.experimental.pallas import tpu as pltpu

R = 8192
C = 128
CH = 512
N = R // CH
D = 8
P = 4


def _body(gu_hbm, gi_hbm, xui_hbm, guo_hbm, gio_hbm,
          ubuf, vbuf, xbuf, uin, vin, uout, vout, xsem):
    def start_in(c):
        s = c % D
        pltpu.make_async_copy(gu_hbm.at[pl.ds(c * CH, CH), :], ubuf.at[s],
                              uin.at[s]).start()
        pltpu.make_async_copy(gi_hbm.at[pl.ds(c * CH, CH), :], vbuf.at[s],
                              vin.at[s]).start()

    def wait_in(c):
        s = c % D
        pltpu.make_async_copy(gu_hbm.at[pl.ds(c * CH, CH), :], ubuf.at[s],
                              uin.at[s]).wait()
        pltpu.make_async_copy(gi_hbm.at[pl.ds(c * CH, CH), :], vbuf.at[s],
                              vin.at[s]).wait()

    def start_out(c):
        s = c % D
        pltpu.make_async_copy(ubuf.at[s], guo_hbm.at[pl.ds(c * CH, CH), :],
                              uout.at[s]).start()
        pltpu.make_async_copy(vbuf.at[s], gio_hbm.at[pl.ds(c * CH, CH), :],
                              vout.at[s]).start()

    def wait_out(c):
        s = c % D
        pltpu.make_async_copy(ubuf.at[s], guo_hbm.at[pl.ds(c * CH, CH), :],
                              uout.at[s]).wait()
        pltpu.make_async_copy(vbuf.at[s], gio_hbm.at[pl.ds(c * CH, CH), :],
                              vout.at[s]).wait()

    for c in range(P):
        start_in(c)

    for c in range(N):
        s = c % D
        wait_in(c)
        start_out(c)
        c2 = c + P
        if c2 < N:
            if c2 >= D:
                wait_out(c2 - D)
            start_in(c2)
        u = ubuf[s]
        v = vbuf[s]
        p = u * v
        s0 = jnp.sum(p[:, :C // 2], axis=1)
        s1 = jnp.sum(p[:, C // 2:], axis=1)
        xbuf[pl.ds(c * CH, CH), :] = jnp.stack([s0, s1], axis=1)

    for c in range(N - D, N):
        wait_out(c)

    cp = pltpu.make_async_copy(xbuf, xui_hbm, xsem)
    cp.start()
    cp.wait()


def kernel(gu, gi):
    B, K = gu.shape
    gu2 = gu.reshape(R, C)
    gi2 = gi.reshape(R, C)
    xui2, guo, gio = pl.pallas_call(
        _body,
        in_specs=[
            pl.BlockSpec(memory_space=pl.ANY),
            pl.BlockSpec(memory_space=pl.ANY),
        ],
        out_specs=[
            pl.BlockSpec(memory_space=pl.ANY),
            pl.BlockSpec(memory_space=pl.ANY),
            pl.BlockSpec(memory_space=pl.ANY),
        ],
        out_shape=[
            jax.ShapeDtypeStruct((R, 2), gu.dtype),
            jax.ShapeDtypeStruct((R, C), gu.dtype),
            jax.ShapeDtypeStruct((R, C), gi.dtype),
        ],
        scratch_shapes=[
            pltpu.VMEM((D, CH, C), jnp.float32),
            pltpu.VMEM((D, CH, C), jnp.float32),
            pltpu.VMEM((R, 2), jnp.float32),
            pltpu.SemaphoreType.DMA((D,)),
            pltpu.SemaphoreType.DMA((D,)),
            pltpu.SemaphoreType.DMA((D,)),
            pltpu.SemaphoreType.DMA((D,)),
            pltpu.SemaphoreType.DMA,
        ],
    )(gu2, gi2)
    return (xui2.reshape(B), guo.reshape(B, K), gio.reshape(B, K))


# speedup vs baseline: 1.6328x; 1.6328x over previous
import jax
import jax.numpy as jnp
from jax.experimental import pallas as pl
from jax.experimental.pallas import tpu as pltpu

B = 16384
K = 64
CH = 1024
N = B // CH
D = 8
P = 4


def _body(gu_hbm, gi_hbm, xui_hbm, guo_hbm, gio_hbm,
          ubuf, vbuf, xbuf, uin, vin, uout, vout, xsem):
    def start_in(c):
        s = c % D
        pltpu.make_async_copy(gu_hbm.at[pl.ds(c * CH, CH), :], ubuf.at[s],
                              uin.at[s]).start()
        pltpu.make_async_copy(gi_hbm.at[pl.ds(c * CH, CH), :], vbuf.at[s],
                              vin.at[s]).start()

    def wait_in(c):
        s = c % D
        pltpu.make_async_copy(gu_hbm.at[pl.ds(c * CH, CH), :], ubuf.at[s],
                              uin.at[s]).wait()
        pltpu.make_async_copy(gi_hbm.at[pl.ds(c * CH, CH), :], vbuf.at[s],
                              vin.at[s]).wait()

    def start_out(c):
        s = c % D
        pltpu.make_async_copy(ubuf.at[s], guo_hbm.at[pl.ds(c * CH, CH), :],
                              uout.at[s]).start()
        pltpu.make_async_copy(vbuf.at[s], gio_hbm.at[pl.ds(c * CH, CH), :],
                              vout.at[s]).start()

    def wait_out(c):
        s = c % D
        pltpu.make_async_copy(ubuf.at[s], guo_hbm.at[pl.ds(c * CH, CH), :],
                              uout.at[s]).wait()
        pltpu.make_async_copy(vbuf.at[s], gio_hbm.at[pl.ds(c * CH, CH), :],
                              vout.at[s]).wait()

    for c in range(P):
        start_in(c)

    for c in range(N):
        s = c % D
        wait_in(c)
        start_out(c)
        c2 = c + P
        if c2 < N:
            if c2 >= D:
                wait_out(c2 - D)
            start_in(c2)
        u = ubuf[s]
        v = vbuf[s]
        xbuf[pl.ds(c * CH, CH)] = jnp.sum(u * v, axis=1)

    for c in range(N - D, N):
        wait_out(c)

    cp = pltpu.make_async_copy(xbuf, xui_hbm, xsem)
    cp.start()
    cp.wait()


def kernel(gu, gi):
    xui, guo, gio = pl.pallas_call(
        _body,
        in_specs=[
            pl.BlockSpec(memory_space=pl.ANY),
            pl.BlockSpec(memory_space=pl.ANY),
        ],
        out_specs=[
            pl.BlockSpec(memory_space=pl.ANY),
            pl.BlockSpec(memory_space=pl.ANY),
            pl.BlockSpec(memory_space=pl.ANY),
        ],
        out_shape=[
            jax.ShapeDtypeStruct((B,), gu.dtype),
            jax.ShapeDtypeStruct((B, K), gu.dtype),
            jax.ShapeDtypeStruct((B, K), gi.dtype),
        ],
        scratch_shapes=[
            pltpu.VMEM((D, CH, K), jnp.float32),
            pltpu.VMEM((D, CH, K), jnp.float32),
            pltpu.VMEM((B,), jnp.float32),
            pltpu.SemaphoreType.DMA((D,)),
            pltpu.SemaphoreType.DMA((D,)),
            pltpu.SemaphoreType.DMA((D,)),
            pltpu.SemaphoreType.DMA((D,)),
            pltpu.SemaphoreType.DMA,
        ],
    )(gu, gi)
    return (xui, guo, gio)
